# baseline (speedup 1.0000x reference)
import jax
import jax.numpy as jnp
from jax import lax
from jax.experimental import pallas as pl
from jax.experimental.pallas import tpu as pltpu
from jax.experimental.pallas import tpu_sc as plsc

VT = 4096
GATHER_WINDOW = 128


def _sc_gather(emb, idx_flat):
    n = idx_flat.shape[0]
    emb_dim = emb.shape[1]
    idx2d = idx_flat.reshape(1, n)

    mesh = plsc.VectorSubcoreMesh(core_axis_name="c", subcore_axis_name="s")

    @pl.kernel(
        out_type=jax.ShapeDtypeStruct((n, emb_dim), emb.dtype),
        mesh=mesh,
        compiler_params=pltpu.CompilerParams(use_tc_tiling_on_sc=False),
    )
    def gather_kernel(emb_hbm, i_hbm, o_hbm):
        def body(i_vmem, o_vmem):
            pltpu.sync_copy(emb_hbm.at[i_vmem.at[0]], o_vmem)

        pltpu.emit_pipeline(
            body,
            grid=(n // GATHER_WINDOW,),
            in_specs=[pl.BlockSpec((1, GATHER_WINDOW), index_map=lambda i: (0, i))],
            out_specs=[pl.BlockSpec((GATHER_WINDOW, emb_dim),
                                    index_map=lambda i: (i, 0))],
            core_axis_name=("c", "s"),
            dimension_semantics=(pltpu.PARALLEL,),
        )(i_hbm, o_hbm)

    return gather_kernel(emb, idx2d)


def _mm1_body(e_ref, w1t_ref, b1_ref, h_ref):
    e = e_ref[...].astype(jnp.bfloat16)
    w1t = w1t_ref[...].astype(jnp.bfloat16)
    h = lax.dot_general(e, w1t, (((1,), (0,)), ((), ())),
                        preferred_element_type=jnp.float32)
    h = h + b1_ref[...][None, :]
    h_ref[...] = jnp.maximum(h, 0.0).astype(jnp.bfloat16)


def _mm1(e, W1t, b1):
    B = e.shape[0]
    HID = W1t.shape[1]
    return pl.pallas_call(
        _mm1_body,
        out_shape=jax.ShapeDtypeStruct((B, HID), jnp.bfloat16),
    )(e, W1t, b1)


def _mm2_body(h_ref, w2_ref, b2_ref, out_ref):
    w2 = w2_ref[...].astype(jnp.bfloat16)
    out = lax.dot_general(w2, h_ref[...], (((1,), (1,)), ((), ())),
                          preferred_element_type=jnp.float32)
    out_ref[...] = out + b2_ref[...][:, None]


def _mm2_t(h, W2, b2):
    B = h.shape[0]
    HID = h.shape[1]
    VOCAB = W2.shape[0]
    grid = (pl.cdiv(VOCAB, VT),)
    return pl.pallas_call(
        _mm2_body,
        grid=grid,
        in_specs=[
            pl.BlockSpec((B, HID), lambda i: (0, 0)),
            pl.BlockSpec((VT, HID), lambda i: (i, 0)),
            pl.BlockSpec((VT,), lambda i: (i,)),
        ],
        out_specs=pl.BlockSpec((VT, B), lambda i: (i, 0)),
        out_shape=jax.ShapeDtypeStruct((VOCAB, B), jnp.float32),
        compiler_params=pltpu.CompilerParams(
            dimension_semantics=("arbitrary",),
            vmem_limit_bytes=60 * 1024 * 1024,
        ),
    )(h, W2, b2)


@jax.jit
def kernel(x, emb, W1, b1, W2, b2):
    B = x.shape[0]
    e_rows = _sc_gather(emb, x.reshape(-1))
    e = e_rows.reshape(B, -1)
    h = _mm1(e, W1.T, b1)
    out_t = _mm2_t(h, W2, b2)
    return out_t.T


# speedup vs baseline: 1.0150x; 1.0150x over previous
import jax
import jax.numpy as jnp
from jax import lax
from jax.experimental import pallas as pl
from jax.experimental.pallas import tpu as pltpu
from jax.experimental.pallas import tpu_sc as plsc

VT = 4096
GATHER_WINDOW = 512


def _sc_gather(emb, idx_flat):
    n = idx_flat.shape[0]
    emb_dim = emb.shape[1]
    idx2d = idx_flat.reshape(1, n)

    mesh = plsc.VectorSubcoreMesh(core_axis_name="c", subcore_axis_name="s")

    @pl.kernel(
        out_type=jax.ShapeDtypeStruct((n, emb_dim), emb.dtype),
        mesh=mesh,
        compiler_params=pltpu.CompilerParams(use_tc_tiling_on_sc=False),
    )
    def gather_kernel(emb_hbm, i_hbm, o_hbm):
        def body(i_vmem, o_vmem):
            pltpu.sync_copy(emb_hbm.at[i_vmem.at[0]], o_vmem)

        pltpu.emit_pipeline(
            body,
            grid=(n // GATHER_WINDOW,),
            in_specs=[pl.BlockSpec((1, GATHER_WINDOW), index_map=lambda i: (0, i))],
            out_specs=[pl.BlockSpec((GATHER_WINDOW, emb_dim),
                                    index_map=lambda i: (i, 0))],
            core_axis_name=("c", "s"),
            dimension_semantics=(pltpu.PARALLEL,),
        )(i_hbm, o_hbm)

    return gather_kernel(emb, idx2d)


def _mm1_body(e_ref, w1t_ref, b1_ref, h_ref):
    e = e_ref[...].astype(jnp.bfloat16)
    w1t = w1t_ref[...].astype(jnp.bfloat16)
    h = lax.dot_general(e, w1t, (((1,), (0,)), ((), ())),
                        preferred_element_type=jnp.float32)
    h = h + b1_ref[...][None, :]
    h_ref[...] = jnp.maximum(h, 0.0).astype(jnp.bfloat16)


def _mm1(e, W1t, b1):
    B = e.shape[0]
    HID = W1t.shape[1]
    return pl.pallas_call(
        _mm1_body,
        out_shape=jax.ShapeDtypeStruct((B, HID), jnp.bfloat16),
    )(e, W1t, b1)


def _mm2_body(h_ref, w2_ref, b2_ref, out_ref):
    w2 = w2_ref[...].astype(jnp.bfloat16)
    out = lax.dot_general(w2, h_ref[...], (((1,), (1,)), ((), ())),
                          preferred_element_type=jnp.float32)
    out_ref[...] = out + b2_ref[...][:, None]


def _mm2_t(h, W2, b2):
    B = h.shape[0]
    HID = h.shape[1]
    VOCAB = W2.shape[0]
    grid = (pl.cdiv(VOCAB, VT),)
    return pl.pallas_call(
        _mm2_body,
        grid=grid,
        in_specs=[
            pl.BlockSpec((B, HID), lambda i: (0, 0)),
            pl.BlockSpec((VT, HID), lambda i: (i, 0)),
            pl.BlockSpec((VT,), lambda i: (i,)),
        ],
        out_specs=pl.BlockSpec((VT, B), lambda i: (i, 0)),
        out_shape=jax.ShapeDtypeStruct((VOCAB, B), jnp.float32),
        compiler_params=pltpu.CompilerParams(
            dimension_semantics=("arbitrary",),
            vmem_limit_bytes=60 * 1024 * 1024,
        ),
    )(h, W2, b2)


@jax.jit
def kernel(x, emb, W1, b1, W2, b2):
    B = x.shape[0]
    e_rows = _sc_gather(emb, x.reshape(-1))
    e = e_rows.reshape(B, -1)
    h = _mm1(e, W1.T, b1)
    out_t = _mm2_t(h, W2, b2)
    return out_t.T


# speedup vs baseline: 1.0190x; 1.0040x over previous
import jax
import jax.numpy as jnp
from jax import lax
from jax.experimental import pallas as pl
from jax.experimental.pallas import tpu as pltpu
from jax.experimental.pallas import tpu_sc as plsc

VT = 4096
GATHER_WINDOW = 1024


def _sc_gather(emb, idx_flat):
    n = idx_flat.shape[0]
    emb_dim = emb.shape[1]
    idx2d = idx_flat.reshape(1, n)

    mesh = plsc.VectorSubcoreMesh(core_axis_name="c", subcore_axis_name="s")

    @pl.kernel(
        out_type=jax.ShapeDtypeStruct((n, emb_dim), emb.dtype),
        mesh=mesh,
        compiler_params=pltpu.CompilerParams(use_tc_tiling_on_sc=False),
    )
    def gather_kernel(emb_hbm, i_hbm, o_hbm):
        def body(i_vmem, o_vmem):
            pltpu.sync_copy(emb_hbm.at[i_vmem.at[0]], o_vmem)

        pltpu.emit_pipeline(
            body,
            grid=(n // GATHER_WINDOW,),
            in_specs=[pl.BlockSpec((1, GATHER_WINDOW), index_map=lambda i: (0, i))],
            out_specs=[pl.BlockSpec((GATHER_WINDOW, emb_dim),
                                    index_map=lambda i: (i, 0))],
            core_axis_name=("c", "s"),
            dimension_semantics=(pltpu.PARALLEL,),
        )(i_hbm, o_hbm)

    return gather_kernel(emb, idx2d)


def _mm1_body(e_ref, w1t_ref, b1_ref, h_ref):
    e = e_ref[...].astype(jnp.bfloat16)
    w1t = w1t_ref[...].astype(jnp.bfloat16)
    h = lax.dot_general(e, w1t, (((1,), (0,)), ((), ())),
                        preferred_element_type=jnp.float32)
    h = h + b1_ref[...][None, :]
    h_ref[...] = jnp.maximum(h, 0.0).astype(jnp.bfloat16)


def _mm1(e, W1t, b1):
    B = e.shape[0]
    HID = W1t.shape[1]
    return pl.pallas_call(
        _mm1_body,
        out_shape=jax.ShapeDtypeStruct((B, HID), jnp.bfloat16),
    )(e, W1t, b1)


def _mm2_body(h_ref, w2_ref, b2_ref, out_ref):
    w2 = w2_ref[...].astype(jnp.bfloat16)
    out = lax.dot_general(w2, h_ref[...], (((1,), (1,)), ((), ())),
                          preferred_element_type=jnp.float32)
    out_ref[...] = out + b2_ref[...][:, None]


def _mm2_t(h, W2, b2):
    B = h.shape[0]
    HID = h.shape[1]
    VOCAB = W2.shape[0]
    grid = (pl.cdiv(VOCAB, VT),)
    return pl.pallas_call(
        _mm2_body,
        grid=grid,
        in_specs=[
            pl.BlockSpec((B, HID), lambda i: (0, 0)),
            pl.BlockSpec((VT, HID), lambda i: (i, 0)),
            pl.BlockSpec((VT,), lambda i: (i,)),
        ],
        out_specs=pl.BlockSpec((VT, B), lambda i: (i, 0)),
        out_shape=jax.ShapeDtypeStruct((VOCAB, B), jnp.float32),
        compiler_params=pltpu.CompilerParams(
            dimension_semantics=("arbitrary",),
            vmem_limit_bytes=60 * 1024 * 1024,
        ),
    )(h, W2, b2)


@jax.jit
def kernel(x, emb, W1, b1, W2, b2):
    B = x.shape[0]
    e_rows = _sc_gather(emb, x.reshape(-1))
    e = e_rows.reshape(B, -1)
    h = _mm1(e, W1.T, b1)
    out_t = _mm2_t(h, W2, b2)
    return out_t.T
